# baseline (speedup 1.0000x reference)
import jax
import jax.numpy as jnp
from jax.experimental import pallas as pl
from jax.experimental.pallas import tpu as pltpu

S_SUB = 13


def kernel(x, embedding):
    seq_len, batch, d_model = x.shape
    r_blk = S_SUB * batch
    rows = seq_len * batch
    x2 = x.reshape(rows, d_model)

    def body(x_ref, emb_ref, out_ref):
        xv = x_ref[...].reshape(S_SUB, batch, d_model)
        out_ref[...] = (xv + emb_ref[...][None, :, :]).reshape(r_blk, d_model)

    out2 = pl.pallas_call(
        body,
        grid=(pl.cdiv(rows, r_blk),),
        in_specs=[
            pl.BlockSpec((r_blk, d_model), lambda i: (i, 0)),
            pl.BlockSpec((batch, d_model), lambda i: (0, 0)),
        ],
        out_specs=pl.BlockSpec((r_blk, d_model), lambda i: (i, 0)),
        out_shape=jax.ShapeDtypeStruct((rows, d_model), x.dtype),
        compiler_params=pltpu.CompilerParams(
            dimension_semantics=("arbitrary",),
        ),
    )(x2, embedding)
    return out2.reshape(seq_len, batch, d_model)


# speedup vs baseline: 1.0006x; 1.0006x over previous
import jax
import jax.numpy as jnp
from jax.experimental import pallas as pl
from jax.experimental.pallas import tpu as pltpu

S_SUB = 14


def kernel(x, embedding):
    seq_len, batch, d_model = x.shape
    r_blk = S_SUB * batch
    rows = seq_len * batch
    x2 = x.reshape(rows, d_model)

    def body(x_ref, emb_ref, out_ref):
        xv = x_ref[...].reshape(S_SUB, batch, d_model)
        out_ref[...] = (xv + emb_ref[...][None, :, :]).reshape(r_blk, d_model)

    out2 = pl.pallas_call(
        body,
        grid=(pl.cdiv(rows, r_blk),),
        in_specs=[
            pl.BlockSpec((r_blk, d_model), lambda i: (i, 0)),
            pl.BlockSpec((batch, d_model), lambda i: (0, 0)),
        ],
        out_specs=pl.BlockSpec((r_blk, d_model), lambda i: (i, 0)),
        out_shape=jax.ShapeDtypeStruct((rows, d_model), x.dtype),
        compiler_params=pltpu.CompilerParams(
            dimension_semantics=("arbitrary",),
        ),
    )(x2, embedding)
    return out2.reshape(seq_len, batch, d_model)
